# baseline (speedup 1.0000x reference)
import functools

import jax
import jax.numpy as jnp
from jax import lax
from jax.experimental import pallas as pl
from jax.experimental.pallas import tpu as pltpu
from jax.experimental.pallas import tpu_sc as plsc

BATCH = 4
SEQ = 4096
D = 2048
P = 10
L = 16
NW = 32
ROWS = BATCH * SEQ
RPW = ROWS // NW
CH = P
NFULL = RPW // CH
TAIL = RPW - NFULL * CH
VPR = D // L
UNROLL = 4


def _sc_body(x_hbm, emb_hbm, out_hbm, emb_rot, bin_v, bout_v,
             sem_i0, sem_i1, sem_o0, sem_o1, sem_t):
    wid = lax.axis_index("s") * 2 + lax.axis_index("c")
    base = wid * (RPW * D)
    ph0 = lax.rem(lax.rem(wid * RPW, SEQ), P)

    sems_in = (sem_i0, sem_i1)
    sems_out = (sem_o0, sem_o1)

    def in_copy(c, slot, nrows=CH):
        return pltpu.make_async_copy(
            x_hbm.at[pl.ds(base + c * (CH * D), nrows * D)],
            bin_v.at[slot, pl.ds(0, nrows * D)], sems_in[slot])

    def out_copy(c, slot, nrows=CH):
        return pltpu.make_async_copy(
            bout_v.at[slot, pl.ds(0, nrows * D)],
            out_hbm.at[pl.ds(base + c * (CH * D), nrows * D)],
            sems_out[slot])

    def compute(slot, nrows=CH):
        @plsc.parallel_loop(0, VPR, 1, unroll=UNROLL)
        def jloop(j):
            off = pl.multiple_of(j * L, L)
            for k in range(nrows):
                bout_v[slot, pl.ds(k * D + off, L)] = (
                    bin_v[slot, pl.ds(k * D + off, L)]
                    + emb_rot[pl.ds(k * D + off, L)])

    for k in range(P):
        src = lax.rem(ph0 + k, P)
        pltpu.make_async_copy(
            emb_hbm.at[pl.ds(src * D, D)],
            emb_rot.at[pl.ds(k * D, D)], sem_t).start()
    for k in range(P):
        pltpu.make_async_copy(
            emb_hbm.at[pl.ds(0, D)], emb_rot.at[pl.ds(0, D)], sem_t).wait()

    in_copy(0, 0).start()
    in_copy(1, 1).start()

    for c in (0, 1):
        slot = c & 1
        in_copy(c, slot).wait()
        compute(slot)
        out_copy(c, slot).start()
        in_copy(c + 2, slot).start()

    def chunk_pair(g, carry):
        for b in range(2):
            c = g * 2 + b
            in_copy(c, b).wait()
            out_copy(c - 2, b).wait()
            compute(b)
            out_copy(c, b).start()
            in_copy(c + 2, b).start()
        return carry

    lax.fori_loop(1, 24, chunk_pair, 0)

    c = 48
    in_copy(c, 0).wait()
    out_copy(c - 2, 0).wait()
    compute(0)
    out_copy(c, 0).start()
    in_copy(50, 0).start()

    c = 49
    in_copy(c, 1).wait()
    out_copy(c - 2, 1).wait()
    compute(1)
    out_copy(c, 1).start()
    in_copy(51, 1, TAIL).start()

    c = 50
    in_copy(c, 0).wait()
    out_copy(c - 2, 0).wait()
    compute(0)
    out_copy(c, 0).start()

    c = 51
    in_copy(c, 1, TAIL).wait()
    out_copy(c - 2, 1).wait()
    compute(1, TAIL)
    out_copy(c, 1, TAIL).start()

    out_copy(50, 0).wait()
    out_copy(51, 1, TAIL).wait()


def kernel(x, embedding):
    mesh = plsc.VectorSubcoreMesh(core_axis_name="c", subcore_axis_name="s")
    fn = functools.partial(
        pl.kernel,
        mesh=mesh,
        out_type=jax.ShapeDtypeStruct((ROWS * D,), jnp.float32),
        scratch_types=[
            pltpu.VMEM((P * D,), jnp.float32),
            pltpu.VMEM((2, CH * D), jnp.float32),
            pltpu.VMEM((2, CH * D), jnp.float32),
            pltpu.SemaphoreType.DMA,
            pltpu.SemaphoreType.DMA,
            pltpu.SemaphoreType.DMA,
            pltpu.SemaphoreType.DMA,
            pltpu.SemaphoreType.DMA,
        ],
    )(_sc_body)
    out = fn(x.reshape(ROWS * D), embedding.astype(jnp.float32).reshape(P * D))
    return out.reshape(x.shape)
